# baseline (speedup 1.0000x reference)
import functools

import jax
import jax.numpy as jnp
from jax.experimental import pallas as pl
from jax.experimental.pallas import tpu as pltpu

_B = 1
_S = 2048
_DM = 768
_DFF = 3072
_E = 8
_RANK = 16
_TOPK = 2
_TS = 512


def _moe_body(h_ref, gw_ref, gb_ref, wi_ref, wo_ref, a_ref, ball_ref, out_ref,
              gwb_ref, wib_ref, wob_ref, ab_ref, ballb_ref):
    def _pack():
        gwb_ref[...] = gw_ref[...].astype(jnp.bfloat16)
        wib_ref[...] = wi_ref[...].astype(jnp.bfloat16)
        wob_ref[...] = wo_ref[...].astype(jnp.bfloat16)
        ab_ref[...] = a_ref[...].astype(jnp.bfloat16)
        ballb_ref[...] = ball_ref[...].astype(jnp.bfloat16)

    _pack()

    h = h_ref[...]
    hb = h.astype(jnp.bfloat16)

    logits = jax.lax.dot_general(
        hb, gwb_ref[...], (((1,), (1,)), ((), ())),
        preferred_element_type=jnp.float32) + gb_ref[...]
    m = jnp.max(logits, axis=-1, keepdims=True)
    ex = jnp.exp(logits - m)
    s = ex / jnp.sum(ex, axis=-1, keepdims=True)
    col = jax.lax.broadcasted_iota(jnp.int32, s.shape, 1)
    v1 = jnp.max(s, axis=-1, keepdims=True)
    i1 = jnp.min(jnp.where(s == v1, col, _E), axis=-1, keepdims=True)
    s2 = jnp.where(col == i1, -jnp.inf, s)
    v2 = jnp.max(s2, axis=-1, keepdims=True)
    i2 = jnp.min(jnp.where(s2 == v2, col, _E), axis=-1, keepdims=True)

    shared = jax.lax.dot_general(
        hb, wib_ref[...], (((1,), (1,)), ((), ())),
        preferred_element_type=jnp.float32)
    p = jax.lax.dot_general(
        hb, ab_ref[...], (((1,), (1,)), ((), ())),
        preferred_element_type=jnp.float32)

    pexp = jax.lax.broadcasted_iota(jnp.int32, p.shape, 1) // _RANK
    q1 = jnp.where(pexp == i1, p, 0.0).astype(jnp.bfloat16)
    q2 = jnp.where(pexp == i2, p, 0.0).astype(jnp.bfloat16)
    l1 = jax.lax.dot_general(
        q1, ballb_ref[...], (((1,), (0,)), ((), ())),
        preferred_element_type=jnp.float32)
    l2 = jax.lax.dot_general(
        q2, ballb_ref[...], (((1,), (0,)), ((), ())),
        preferred_element_type=jnp.float32)

    acc = v1 * jnp.maximum(shared + l1, 0.0) + v2 * jnp.maximum(shared + l2, 0.0)

    out_ref[...] = jax.lax.dot_general(
        acc.astype(jnp.bfloat16), wob_ref[...], (((1,), (1,)), ((), ())),
        preferred_element_type=jnp.float32)


@functools.partial(jax.jit, static_argnames=())
def _moe(h2d, gate_w, gate_b2d, wi_w, wo_w, a_all, ball):
    grid = (_S // _TS,)
    return pl.pallas_call(
        _moe_body,
        grid=grid,
        in_specs=[
            pl.BlockSpec((_TS, _DM), lambda i: (i, 0)),
            pl.BlockSpec((_E, _DM), lambda i: (0, 0)),
            pl.BlockSpec((1, _E), lambda i: (0, 0)),
            pl.BlockSpec((_DFF, _DM), lambda i: (0, 0)),
            pl.BlockSpec((_DM, _DFF), lambda i: (0, 0)),
            pl.BlockSpec((_E * _RANK, _DM), lambda i: (0, 0)),
            pl.BlockSpec((_E * _RANK, _DFF), lambda i: (0, 0)),
        ],
        out_specs=pl.BlockSpec((_TS, _DM), lambda i: (i, 0)),
        out_shape=jax.ShapeDtypeStruct((_S, _DM), jnp.float32),
        scratch_shapes=[
            pltpu.VMEM((_E, _DM), jnp.bfloat16),
            pltpu.VMEM((_DFF, _DM), jnp.bfloat16),
            pltpu.VMEM((_DM, _DFF), jnp.bfloat16),
            pltpu.VMEM((_E * _RANK, _DM), jnp.bfloat16),
            pltpu.VMEM((_E * _RANK, _DFF), jnp.bfloat16),
        ],
        compiler_params=pltpu.CompilerParams(
            dimension_semantics=("parallel",),
        ),
    )(h2d, gate_w, gate_b2d, wi_w, wo_w, a_all, ball)


def kernel(hidden_states, gate_w, gate_b, wi_w, wo_w, lora_A, lora_B):
    h2d = hidden_states.reshape(_S, _DM)
    gate_b2d = gate_b.reshape(1, _E)
    a_all = lora_A.reshape(_E * _RANK, _DM)
    ball = jnp.transpose(lora_B, (0, 2, 1)).reshape(_E * _RANK, _DFF)
    out = _moe(h2d, gate_w, gate_b2d, wi_w, wo_w, a_all, ball)
    return out.reshape(_B, _S, _DM)


# speedup vs baseline: 1.0080x; 1.0080x over previous
import functools

import jax
import jax.numpy as jnp
from jax.experimental import pallas as pl
from jax.experimental.pallas import tpu as pltpu

_B = 1
_S = 2048
_DM = 768
_DFF = 3072
_E = 8
_RANK = 16
_TOPK = 2
_TS = 512


def _moe_body(h_ref, gw_ref, gb_ref, wi_ref, wo_hbm, a_ref, ball_ref, out_ref,
              gwb_ref, wib_ref, wob_ref, ab_ref, ballb_ref, wof_ref, wo_sem):
    @pl.when(pl.program_id(0) == 0)
    def _pack():
        pltpu.make_async_copy(wo_hbm, wof_ref, wo_sem).start()
        gwb_ref[...] = gw_ref[...].astype(jnp.bfloat16)
        wib_ref[...] = wi_ref[...].astype(jnp.bfloat16)
        ab_ref[...] = a_ref[...].astype(jnp.bfloat16)
        ballb_ref[...] = ball_ref[...].astype(jnp.bfloat16)

    h = h_ref[...]
    hb = h.astype(jnp.bfloat16)

    logits = jax.lax.dot_general(
        hb, gwb_ref[...], (((1,), (1,)), ((), ())),
        preferred_element_type=jnp.float32) + gb_ref[...]
    m = jnp.max(logits, axis=-1, keepdims=True)
    ex = jnp.exp(logits - m)
    s = ex / jnp.sum(ex, axis=-1, keepdims=True)
    col = jax.lax.broadcasted_iota(jnp.int32, s.shape, 1)
    v1 = jnp.max(s, axis=-1, keepdims=True)
    i1 = jnp.min(jnp.where(s == v1, col, _E), axis=-1, keepdims=True)
    s2 = jnp.where(col == i1, -jnp.inf, s)
    v2 = jnp.max(s2, axis=-1, keepdims=True)
    i2 = jnp.min(jnp.where(s2 == v2, col, _E), axis=-1, keepdims=True)

    shared = jax.lax.dot_general(
        hb, wib_ref[...], (((1,), (1,)), ((), ())),
        preferred_element_type=jnp.float32)
    p = jax.lax.dot_general(
        hb, ab_ref[...], (((1,), (1,)), ((), ())),
        preferred_element_type=jnp.float32)

    pexp = jax.lax.broadcasted_iota(jnp.int32, p.shape, 1) // _RANK
    q1 = jnp.where(pexp == i1, p, 0.0).astype(jnp.bfloat16)
    q2 = jnp.where(pexp == i2, p, 0.0).astype(jnp.bfloat16)
    l1 = jax.lax.dot_general(
        q1, ballb_ref[...], (((1,), (0,)), ((), ())),
        preferred_element_type=jnp.float32)
    l2 = jax.lax.dot_general(
        q2, ballb_ref[...], (((1,), (0,)), ((), ())),
        preferred_element_type=jnp.float32)

    acc = v1 * jnp.maximum(shared + l1, 0.0) + v2 * jnp.maximum(shared + l2, 0.0)

    @pl.when(pl.program_id(0) == 0)
    def _finish_wo():
        pltpu.make_async_copy(wo_hbm, wof_ref, wo_sem).wait()
        wob_ref[...] = wof_ref[...].astype(jnp.bfloat16)

    out_ref[...] = jax.lax.dot_general(
        acc.astype(jnp.bfloat16), wob_ref[...], (((1,), (1,)), ((), ())),
        preferred_element_type=jnp.float32)


@functools.partial(jax.jit, static_argnames=())
def _moe(h2d, gate_w, gate_b2d, wi_w, wo_w, a_all, ball):
    grid = (_S // _TS,)
    return pl.pallas_call(
        _moe_body,
        grid=grid,
        in_specs=[
            pl.BlockSpec((_TS, _DM), lambda i: (i, 0)),
            pl.BlockSpec((_E, _DM), lambda i: (0, 0)),
            pl.BlockSpec((1, _E), lambda i: (0, 0)),
            pl.BlockSpec((_DFF, _DM), lambda i: (0, 0)),
            pl.BlockSpec(memory_space=pl.ANY),
            pl.BlockSpec((_E * _RANK, _DM), lambda i: (0, 0)),
            pl.BlockSpec((_E * _RANK, _DFF), lambda i: (0, 0)),
        ],
        out_specs=pl.BlockSpec((_TS, _DM), lambda i: (i, 0)),
        out_shape=jax.ShapeDtypeStruct((_S, _DM), jnp.float32),
        scratch_shapes=[
            pltpu.VMEM((_E, _DM), jnp.bfloat16),
            pltpu.VMEM((_DFF, _DM), jnp.bfloat16),
            pltpu.VMEM((_DM, _DFF), jnp.bfloat16),
            pltpu.VMEM((_E * _RANK, _DM), jnp.bfloat16),
            pltpu.VMEM((_E * _RANK, _DFF), jnp.bfloat16),
            pltpu.VMEM((_DM, _DFF), jnp.float32),
            pltpu.SemaphoreType.DMA,
        ],
        compiler_params=pltpu.CompilerParams(
            dimension_semantics=("arbitrary",),
        ),
    )(h2d, gate_w, gate_b2d, wi_w, wo_w, a_all, ball)


def kernel(hidden_states, gate_w, gate_b, wi_w, wo_w, lora_A, lora_B):
    h2d = hidden_states.reshape(_S, _DM)
    gate_b2d = gate_b.reshape(1, _E)
    a_all = lora_A.reshape(_E * _RANK, _DM)
    ball = jnp.transpose(lora_B, (0, 2, 1)).reshape(_E * _RANK, _DFF)
    out = _moe(h2d, gate_w, gate_b2d, wi_w, wo_w, a_all, ball)
    return out.reshape(_B, _S, _DM)


# speedup vs baseline: 1.0145x; 1.0064x over previous
import functools

import jax
import jax.numpy as jnp
from jax.experimental import pallas as pl
from jax.experimental.pallas import tpu as pltpu

_B = 1
_S = 2048
_DM = 768
_DFF = 3072
_E = 8
_RANK = 16
_TOPK = 2
_TS = 512


def _moe_body(h_ref, gw_ref, gb_ref, wi_hbm, wo_hbm, a_ref, ball_ref, out_ref,
              gwb_ref, wib_ref, wob_ref, ab_ref, ballb_ref, wif_ref, wof_ref,
              wi_sem, wo_sem):
    @pl.when(pl.program_id(0) == 0)
    def _pack():
        pltpu.make_async_copy(wi_hbm, wif_ref, wi_sem).start()
        pltpu.make_async_copy(wo_hbm, wof_ref, wo_sem).start()
        gwb_ref[...] = gw_ref[...].astype(jnp.bfloat16)
        ab_ref[...] = a_ref[...].astype(jnp.bfloat16)
        ballb_ref[...] = ball_ref[...].astype(jnp.bfloat16)

    h = h_ref[...]
    hb = h.astype(jnp.bfloat16)

    logits = jax.lax.dot_general(
        hb, gwb_ref[...], (((1,), (1,)), ((), ())),
        preferred_element_type=jnp.float32) + gb_ref[...]
    m = jnp.max(logits, axis=-1, keepdims=True)
    ex = jnp.exp(logits - m)
    s = ex / jnp.sum(ex, axis=-1, keepdims=True)
    col = jax.lax.broadcasted_iota(jnp.int32, s.shape, 1)
    v1 = jnp.max(s, axis=-1, keepdims=True)
    i1 = jnp.min(jnp.where(s == v1, col, _E), axis=-1, keepdims=True)
    s2 = jnp.where(col == i1, -jnp.inf, s)
    v2 = jnp.max(s2, axis=-1, keepdims=True)
    i2 = jnp.min(jnp.where(s2 == v2, col, _E), axis=-1, keepdims=True)

    p = jax.lax.dot_general(
        hb, ab_ref[...], (((1,), (1,)), ((), ())),
        preferred_element_type=jnp.float32)

    pexp = jax.lax.broadcasted_iota(jnp.int32, p.shape, 1) // _RANK
    q1 = jnp.where(pexp == i1, p, 0.0).astype(jnp.bfloat16)
    q2 = jnp.where(pexp == i2, p, 0.0).astype(jnp.bfloat16)
    l1 = jax.lax.dot_general(
        q1, ballb_ref[...], (((1,), (0,)), ((), ())),
        preferred_element_type=jnp.float32)
    l2 = jax.lax.dot_general(
        q2, ballb_ref[...], (((1,), (0,)), ((), ())),
        preferred_element_type=jnp.float32)

    @pl.when(pl.program_id(0) == 0)
    def _finish_wi():
        pltpu.make_async_copy(wi_hbm, wif_ref, wi_sem).wait()
        wib_ref[...] = wif_ref[...].astype(jnp.bfloat16)

    shared = jax.lax.dot_general(
        hb, wib_ref[...], (((1,), (1,)), ((), ())),
        preferred_element_type=jnp.float32)

    acc = v1 * jnp.maximum(shared + l1, 0.0) + v2 * jnp.maximum(shared + l2, 0.0)

    @pl.when(pl.program_id(0) == 0)
    def _finish_wo():
        pltpu.make_async_copy(wo_hbm, wof_ref, wo_sem).wait()
        wob_ref[...] = wof_ref[...].astype(jnp.bfloat16)

    out_ref[...] = jax.lax.dot_general(
        acc.astype(jnp.bfloat16), wob_ref[...], (((1,), (1,)), ((), ())),
        preferred_element_type=jnp.float32)


@functools.partial(jax.jit, static_argnames=())
def _moe(h2d, gate_w, gate_b2d, wi_w, wo_w, a_all, ball):
    grid = (_S // _TS,)
    return pl.pallas_call(
        _moe_body,
        grid=grid,
        in_specs=[
            pl.BlockSpec((_TS, _DM), lambda i: (i, 0)),
            pl.BlockSpec((_E, _DM), lambda i: (0, 0)),
            pl.BlockSpec((1, _E), lambda i: (0, 0)),
            pl.BlockSpec(memory_space=pl.ANY),
            pl.BlockSpec(memory_space=pl.ANY),
            pl.BlockSpec((_E * _RANK, _DM), lambda i: (0, 0)),
            pl.BlockSpec((_E * _RANK, _DFF), lambda i: (0, 0)),
        ],
        out_specs=pl.BlockSpec((_TS, _DM), lambda i: (i, 0)),
        out_shape=jax.ShapeDtypeStruct((_S, _DM), jnp.float32),
        scratch_shapes=[
            pltpu.VMEM((_E, _DM), jnp.bfloat16),
            pltpu.VMEM((_DFF, _DM), jnp.bfloat16),
            pltpu.VMEM((_DM, _DFF), jnp.bfloat16),
            pltpu.VMEM((_E * _RANK, _DM), jnp.bfloat16),
            pltpu.VMEM((_E * _RANK, _DFF), jnp.bfloat16),
            pltpu.VMEM((_DFF, _DM), jnp.float32),
            pltpu.VMEM((_DM, _DFF), jnp.float32),
            pltpu.SemaphoreType.DMA,
            pltpu.SemaphoreType.DMA,
        ],
        compiler_params=pltpu.CompilerParams(
            dimension_semantics=("arbitrary",),
        ),
    )(h2d, gate_w, gate_b2d, wi_w, wo_w, a_all, ball)


def kernel(hidden_states, gate_w, gate_b, wi_w, wo_w, lora_A, lora_B):
    h2d = hidden_states.reshape(_S, _DM)
    gate_b2d = gate_b.reshape(1, _E)
    a_all = lora_A.reshape(_E * _RANK, _DM)
    ball = jnp.transpose(lora_B, (0, 2, 1)).reshape(_E * _RANK, _DFF)
    out = _moe(h2d, gate_w, gate_b2d, wi_w, wo_w, a_all, ball)
    return out.reshape(_B, _S, _DM)
